# baseline (speedup 1.0000x reference)
import functools

import jax
import jax.numpy as jnp
import numpy as np
from jax.experimental import pallas as pl
from jax.experimental.pallas import tpu as pltpu
from jax.experimental.pallas import tpu_sc as plsc

EPS = float(np.finfo(np.float32).eps)
N = 2048
C = 768
E = 8
FFN = 2048
TILE = 128
NUM_TILES = 40
MROWS = NUM_TILES + 8
PADDED = NUM_TILES * TILE
HALF = C // 2
SCWIN = 128


def _router_body(x_ref, gate_ref, xnb_ref, dest_ref, w_ref, meta_ref):
    x = x_ref[...]
    xn = x * jax.lax.rsqrt(jnp.mean(x * x, axis=1, keepdims=True) + EPS)
    xnb_ref[...] = xn
    logits = jax.lax.dot_general(
        xn, gate_ref[...], (((1,), (1,)), ((), ())),
        preferred_element_type=jnp.float32)
    m1 = jnp.max(logits, axis=1, keepdims=True)
    ex = jnp.exp(logits - m1)
    scores = ex / jnp.sum(ex, axis=1, keepdims=True)
    iota_e = jax.lax.broadcasted_iota(jnp.int32, (N, E), 1)
    tri_e = (jax.lax.broadcasted_iota(jnp.int32, (E, E), 0)
             <= jax.lax.broadcasted_iota(jnp.int32, (E, E), 1)).astype(jnp.float32)
    eq1 = (logits == m1).astype(jnp.float32)
    cs1 = jax.lax.dot_general(eq1, tri_e, (((1,), (0,)), ((), ())),
                              preferred_element_type=jnp.float32)
    oh1 = eq1 * (cs1 == 1.0).astype(jnp.float32)
    masked = jnp.where(oh1 > 0.0, -jnp.inf, logits)
    m2 = jnp.max(masked, axis=1, keepdims=True)
    eq2 = (masked == m2).astype(jnp.float32)
    cs2 = jax.lax.dot_general(eq2, tri_e, (((1,), (0,)), ((), ())),
                              preferred_element_type=jnp.float32)
    oh2 = eq2 * (cs2 == 1.0).astype(jnp.float32)
    w1 = jnp.sum(scores * oh1, axis=1, keepdims=True)
    w2 = jnp.sum(scores * oh2, axis=1, keepdims=True)
    s = w1 + w2 + 1e-6
    iota2 = jax.lax.broadcasted_iota(jnp.int32, (N, 2), 1)
    w_ref[...] = jnp.where(iota2 == 0, w1 / s, w2 / s)
    e1 = jnp.sum(jnp.where(oh1 > 0.0, iota_e, 0), axis=1, keepdims=True)
    e2 = jnp.sum(jnp.where(oh2 > 0.0, iota_e, 0), axis=1, keepdims=True)
    iota16 = jax.lax.broadcasted_iota(jnp.int32, (N, 2 * E), 1)
    oh16 = (jnp.equal(iota16, e1) | jnp.equal(iota16, e2 + E)).astype(jnp.bfloat16)
    ltri = (jax.lax.broadcasted_iota(jnp.int32, (N, N), 1)
            < jax.lax.broadcasted_iota(jnp.int32, (N, N), 0)).astype(jnp.bfloat16)
    run = jax.lax.dot_general(ltri, oh16, (((1,), (0,)), ((), ())),
                              preferred_element_type=jnp.float32)
    cnt1 = jnp.sum(oh1, axis=0, keepdims=True)
    cnt2 = jnp.sum(oh2, axis=0, keepdims=True)
    cnt = cnt1 + cnt2
    ptiles = jnp.ceil(cnt * (1.0 / TILE))
    stri_e = (jax.lax.broadcasted_iota(jnp.int32, (E, E), 0)
              < jax.lax.broadcasted_iota(jnp.int32, (E, E), 1)).astype(jnp.float32)
    base_tile = jax.lax.dot_general(ptiles, stri_e, (((1,), (0,)), ((), ())),
                                    preferred_element_type=jnp.float32)
    base_rows = base_tile * float(TILE)
    rank1 = jnp.sum(run[:, :E] * oh1, axis=1, keepdims=True)
    rank2 = (jnp.sum(run[:, E:] * oh2, axis=1, keepdims=True)
             + jnp.sum(cnt1 * oh2, axis=1, keepdims=True))
    dest1 = jnp.sum(base_rows * oh1, axis=1, keepdims=True) + rank1
    dest2 = jnp.sum(base_rows * oh2, axis=1, keepdims=True) + rank2
    iota4 = jax.lax.broadcasted_iota(jnp.int32, (N, 4), 1)
    d1 = 2.0 * dest1
    d2 = 2.0 * dest2
    dest_ref[...] = jnp.where(
        iota4 == 0, d1, jnp.where(iota4 == 1, d1 + 1.0,
                                  jnp.where(iota4 == 2, d2, d2 + 1.0))).astype(jnp.int32)
    trow = jax.lax.broadcasted_iota(jnp.int32, (MROWS, E), 0).astype(jnp.float32)
    te = jnp.sum((base_tile <= trow).astype(jnp.float32), axis=1, keepdims=True) - 1.0
    n_active = jnp.sum(ptiles)
    mrow = jax.lax.broadcasted_iota(jnp.int32, (MROWS, 1), 0)
    meta_ref[...] = jnp.where(mrow == NUM_TILES, n_active, te).astype(jnp.int32)


def _router(xf, gate_w):
    return pl.pallas_call(
        _router_body,
        out_shape=(
            jax.ShapeDtypeStruct((N, C), jnp.float32),
            jax.ShapeDtypeStruct((N, 4), jnp.int32),
            jax.ShapeDtypeStruct((N, 2), jnp.float32),
            jax.ShapeDtypeStruct((MROWS, 1), jnp.int32),
        ),
    )(xf, gate_w)


def _sc_scatter(xnb2, dest_cat):
    mesh = plsc.VectorSubcoreMesh(core_axis_name="core", subcore_axis_name="subcore")
    nwin = 2 * N // SCWIN

    @functools.partial(
        pl.kernel,
        out_type=jax.ShapeDtypeStruct((2 * PADDED, HALF), jnp.float32),
        mesh=mesh)
    def k(x_hbm, i_hbm, o_hbm):
        def body(x_vmem, i_vmem):
            pltpu.sync_copy(x_vmem, o_hbm.at[i_vmem.at[0]])

        pltpu.emit_pipeline(
            body,
            grid=(4 * N // SCWIN,),
            in_specs=[
                pl.BlockSpec((SCWIN, HALF), lambda i: (i % nwin, 0)),
                pl.BlockSpec((1, SCWIN), lambda i: (0, i)),
            ],
            out_specs=[],
            core_axis_name=("core", "subcore"),
            dimension_semantics=(pltpu.PARALLEL,),
        )(x_hbm, i_hbm)

    return k(xnb2, dest_cat)


def _sc_gather(ys2, dest_cat):
    mesh = plsc.VectorSubcoreMesh(core_axis_name="core", subcore_axis_name="subcore")

    @functools.partial(
        pl.kernel,
        out_type=jax.ShapeDtypeStruct((4 * N, HALF), jnp.float32),
        mesh=mesh)
    def k(y_hbm, i_hbm, o_hbm):
        def body(i_vmem, o_vmem):
            pltpu.sync_copy(y_hbm.at[i_vmem.at[0]], o_vmem)

        pltpu.emit_pipeline(
            body,
            grid=(4 * N // SCWIN,),
            in_specs=[pl.BlockSpec((1, SCWIN), lambda i: (0, i))],
            out_specs=[pl.BlockSpec((SCWIN, HALF), lambda i: (i, 0))],
            core_axis_name=("core", "subcore"),
            dimension_semantics=(pltpu.PARALLEL,),
        )(i_hbm, o_hbm)

    return k(ys2, dest_cat)


def _mlp_body(te_ref, xs_ref, w1_ref, w2_ref, ys_ref):
    i = pl.program_id(0)

    @pl.when(i < te_ref[NUM_TILES])
    def _():
        h = jax.lax.dot_general(
            xs_ref[...].astype(jnp.bfloat16), w1_ref[0], (((1,), (1,)), ((), ())),
            preferred_element_type=jnp.float32)
        h = jnp.square(jnp.maximum(h, 0.0)).astype(jnp.bfloat16)
        ys_ref[...] = jax.lax.dot_general(
            h, w2_ref[0], (((1,), (1,)), ((), ())),
            preferred_element_type=jnp.float32)


def _mlp(meta, xs, w1b, w2b):
    grid_spec = pltpu.PrefetchScalarGridSpec(
        num_scalar_prefetch=1,
        grid=(NUM_TILES,),
        in_specs=[
            pl.BlockSpec((TILE, C), lambda i, te: (i, 0)),
            pl.BlockSpec((1, FFN, C), lambda i, te: (te[i], 0, 0)),
            pl.BlockSpec((1, C, FFN), lambda i, te: (te[i], 0, 0)),
        ],
        out_specs=pl.BlockSpec((TILE, C), lambda i, te: (i, 0)),
    )
    return pl.pallas_call(
        _mlp_body,
        grid_spec=grid_spec,
        out_shape=jax.ShapeDtypeStruct((PADDED, C), jnp.float32),
    )(meta, xs, w1b, w2b)


def _combine_body(x_ref, y0_ref, y1_ref, w_ref, sg_ref, kpw_ref, kpb_ref,
                  vpw_ref, vpb_ref, o_ref):
    x = x_ref[...]
    xn = x * jax.lax.rsqrt(jnp.mean(x * x, axis=1, keepdims=True) + EPS)
    g = jax.nn.sigmoid(jnp.sum(xn * sg_ref[...], axis=1, keepdims=True))
    x2 = xn * jax.lax.rsqrt(jnp.mean(xn * xn, axis=1, keepdims=True) + EPS)
    kk = jax.lax.dot_general(
        x2.astype(jnp.bfloat16), kpw_ref[...], (((1,), (1,)), ((), ())),
        preferred_element_type=jnp.float32) + kpb_ref[...]
    kk = jnp.square(jnp.maximum(kk, 0.0))
    sh = jax.lax.dot_general(
        kk.astype(jnp.bfloat16), vpw_ref[...], (((1,), (1,)), ((), ())),
        preferred_element_type=jnp.float32) + vpb_ref[...]
    w = w_ref[...]
    o_ref[...] = (w[:, 0:1] * y0_ref[...] + w[:, 1:2] * y1_ref[...] + g * sh)


def _combine(xf, y01, w, sg_w, kpw, kpb, vpw, vpb):
    rows = 256
    nblk = N // rows
    return pl.pallas_call(
        _combine_body,
        grid=(nblk,),
        in_specs=[
            pl.BlockSpec((rows, C), lambda i: (i, 0)),
            pl.BlockSpec((rows, C), lambda i: (i, 0)),
            pl.BlockSpec((rows, C), lambda i: (i + nblk, 0)),
            pl.BlockSpec((rows, 2), lambda i: (i, 0)),
            pl.BlockSpec((1, C), lambda i: (0, 0)),
            pl.BlockSpec((FFN, C), lambda i: (0, 0)),
            pl.BlockSpec((1, FFN), lambda i: (0, 0)),
            pl.BlockSpec((C, FFN), lambda i: (0, 0)),
            pl.BlockSpec((1, C), lambda i: (0, 0)),
        ],
        out_specs=pl.BlockSpec((rows, C), lambda i: (i, 0)),
        out_shape=jax.ShapeDtypeStruct((N, C), jnp.float32),
    )(xf, y01, y01, w, sg_w, kpw, kpb, vpw, vpb)


def kernel(x, gate_w, W1, W2, sg_w, kp_w, kp_b, vp_w, vp_b):
    Bs, Ts, Cs = x.shape
    xf = x.reshape(Ts, Cs)
    xnb, dest, w, meta = _router(xf, gate_w)
    dest_cat = jnp.concatenate(
        [dest[:, :2].reshape(-1), dest[:, 2:].reshape(-1)]).reshape(1, 4 * N)
    xs2 = _sc_scatter(xnb.reshape(2 * N, HALF), dest_cat)
    ys = _mlp(meta.reshape(MROWS), xs2.reshape(PADDED, C),
              W1.astype(jnp.bfloat16), W2.astype(jnp.bfloat16))
    y01 = _sc_gather(ys.reshape(2 * PADDED, HALF), dest_cat).reshape(2 * N, C)
    out = _combine(xf, y01, w, sg_w,
                   kp_w.astype(jnp.bfloat16), kp_b.reshape(1, FFN),
                   vp_w.astype(jnp.bfloat16), vp_b.reshape(1, C))
    return out.reshape(Bs, Ts, Cs)


# speedup vs baseline: 1.4388x; 1.4388x over previous
import functools

import jax
import jax.numpy as jnp
import numpy as np
from jax.experimental import pallas as pl
from jax.experimental.pallas import tpu as pltpu
from jax.experimental.pallas import tpu_sc as plsc

EPS = float(np.finfo(np.float32).eps)
N = 2048
C = 768
E = 8
FFN = 2048
TILE = 256
NUM_TILES = 24
MROWS = NUM_TILES + 8
PADDED = NUM_TILES * TILE
NSUB = 32
RPW = 2 * N // NSUB


def _router_body(x_ref, gate_ref, xnb_ref, dest_ref, w_ref, meta_ref):
    x = x_ref[...]
    xn = x * jax.lax.rsqrt(jnp.mean(x * x, axis=1, keepdims=True) + EPS)
    xnb_ref[...] = xn
    logits = jax.lax.dot_general(
        xn, gate_ref[...], (((1,), (1,)), ((), ())),
        preferred_element_type=jnp.float32)
    m1 = jnp.max(logits, axis=1, keepdims=True)
    ex = jnp.exp(logits - m1)
    scores = ex / jnp.sum(ex, axis=1, keepdims=True)
    iota_e = jax.lax.broadcasted_iota(jnp.int32, (N, E), 1)
    tri_e = (jax.lax.broadcasted_iota(jnp.int32, (E, E), 0)
             <= jax.lax.broadcasted_iota(jnp.int32, (E, E), 1)).astype(jnp.float32)
    eq1 = (logits == m1).astype(jnp.float32)
    cs1 = jax.lax.dot_general(eq1, tri_e, (((1,), (0,)), ((), ())),
                              preferred_element_type=jnp.float32)
    oh1 = eq1 * (cs1 == 1.0).astype(jnp.float32)
    masked = jnp.where(oh1 > 0.0, -jnp.inf, logits)
    m2 = jnp.max(masked, axis=1, keepdims=True)
    eq2 = (masked == m2).astype(jnp.float32)
    cs2 = jax.lax.dot_general(eq2, tri_e, (((1,), (0,)), ((), ())),
                              preferred_element_type=jnp.float32)
    oh2 = eq2 * (cs2 == 1.0).astype(jnp.float32)
    w1 = jnp.sum(scores * oh1, axis=1, keepdims=True)
    w2 = jnp.sum(scores * oh2, axis=1, keepdims=True)
    s = w1 + w2 + 1e-6
    iota2 = jax.lax.broadcasted_iota(jnp.int32, (N, 2), 1)
    w_ref[...] = jnp.where(iota2 == 0, w1 / s, w2 / s)
    e1 = jnp.sum(jnp.where(oh1 > 0.0, iota_e, 0), axis=1, keepdims=True)
    e2 = jnp.sum(jnp.where(oh2 > 0.0, iota_e, 0), axis=1, keepdims=True)
    iota16 = jax.lax.broadcasted_iota(jnp.int32, (N, 2 * E), 1)
    oh16 = (jnp.equal(iota16, e1) | jnp.equal(iota16, e2 + E)).astype(jnp.bfloat16)
    ltri = (jax.lax.broadcasted_iota(jnp.int32, (N, N), 1)
            < jax.lax.broadcasted_iota(jnp.int32, (N, N), 0)).astype(jnp.bfloat16)
    run = jax.lax.dot_general(ltri, oh16, (((1,), (0,)), ((), ())),
                              preferred_element_type=jnp.float32)
    cnt1 = jnp.sum(oh1, axis=0, keepdims=True)
    cnt2 = jnp.sum(oh2, axis=0, keepdims=True)
    cnt = cnt1 + cnt2
    ptiles = jnp.ceil(cnt * (1.0 / TILE))
    stri_e = (jax.lax.broadcasted_iota(jnp.int32, (E, E), 0)
              < jax.lax.broadcasted_iota(jnp.int32, (E, E), 1)).astype(jnp.float32)
    base_tile = jax.lax.dot_general(ptiles, stri_e, (((1,), (0,)), ((), ())),
                                    preferred_element_type=jnp.float32)
    base_rows = base_tile * float(TILE)
    rank1 = jnp.sum(run[:, :E] * oh1, axis=1, keepdims=True)
    rank2 = (jnp.sum(run[:, E:] * oh2, axis=1, keepdims=True)
             + jnp.sum(cnt1 * oh2, axis=1, keepdims=True))
    dest1 = jnp.sum(base_rows * oh1, axis=1, keepdims=True) + rank1
    dest2 = jnp.sum(base_rows * oh2, axis=1, keepdims=True) + rank2
    dest_ref[...] = jnp.where(iota2 == 0, dest1, dest2).astype(jnp.int32)
    trow = jax.lax.broadcasted_iota(jnp.int32, (MROWS, E), 0).astype(jnp.float32)
    te = jnp.sum((base_tile <= trow).astype(jnp.float32), axis=1, keepdims=True) - 1.0
    n_active = jnp.sum(ptiles)
    mrow = jax.lax.broadcasted_iota(jnp.int32, (MROWS, 1), 0)
    meta_ref[...] = jnp.where(mrow == NUM_TILES, n_active, te).astype(jnp.int32)


def _router(xf, gate_w):
    return pl.pallas_call(
        _router_body,
        out_shape=(
            jax.ShapeDtypeStruct((N, C), jnp.float32),
            jax.ShapeDtypeStruct((N, 2), jnp.int32),
            jax.ShapeDtypeStruct((N, 2), jnp.float32),
            jax.ShapeDtypeStruct((MROWS, 1), jnp.int32),
        ),
    )(xf, gate_w)


def _sc_scatter(xnb, dest_cat):
    mesh = plsc.VectorSubcoreMesh(core_axis_name="c", subcore_axis_name="s")

    @functools.partial(
        pl.kernel,
        out_type=jax.ShapeDtypeStruct((PADDED, C), jnp.float32),
        mesh=mesh,
        scratch_types=[
            pltpu.VMEM((RPW,), jnp.int32),
            pltpu.VMEM((RPW, C), jnp.float32),
            pltpu.SemaphoreType.DMA,
        ])
    def k(x_hbm, i_hbm, o_hbm, idx_v, rows_v, sem):
        wid = jax.lax.axis_index("s") * 2 + jax.lax.axis_index("c")
        base = wid * RPW
        pltpu.sync_copy(i_hbm.at[pl.ds(base, RPW)], idx_v)
        pltpu.sync_copy(x_hbm.at[pl.ds(base % N, RPW)], rows_v)
        pltpu.async_copy(rows_v, o_hbm.at[idx_v], sem).wait()

    return k(xnb, dest_cat)


def _sc_gather(ys, dest_cat):
    mesh = plsc.VectorSubcoreMesh(core_axis_name="c", subcore_axis_name="s")

    @functools.partial(
        pl.kernel,
        out_type=jax.ShapeDtypeStruct((2 * N, C), jnp.float32),
        mesh=mesh,
        scratch_types=[
            pltpu.VMEM((RPW,), jnp.int32),
            pltpu.VMEM((RPW, C), jnp.float32),
            pltpu.SemaphoreType.DMA,
        ])
    def k(y_hbm, i_hbm, o_hbm, idx_v, rows_v, sem):
        wid = jax.lax.axis_index("s") * 2 + jax.lax.axis_index("c")
        base = wid * RPW
        pltpu.sync_copy(i_hbm.at[pl.ds(base, RPW)], idx_v)
        pltpu.async_copy(y_hbm.at[idx_v], rows_v, sem).wait()
        pltpu.sync_copy(rows_v, o_hbm.at[pl.ds(base, RPW)])

    return k(ys, dest_cat)


def _mlp_body(te_ref, xs_ref, w1_ref, w2_ref, ys_ref):
    i = pl.program_id(0)

    @pl.when(i < te_ref[NUM_TILES])
    def _():
        h = jax.lax.dot_general(
            xs_ref[...].astype(jnp.bfloat16), w1_ref[0], (((1,), (1,)), ((), ())),
            preferred_element_type=jnp.float32)
        h = jnp.square(jnp.maximum(h, 0.0)).astype(jnp.bfloat16)
        ys_ref[...] = jax.lax.dot_general(
            h, w2_ref[0], (((1,), (1,)), ((), ())),
            preferred_element_type=jnp.float32)


def _mlp(meta, xs, w1b, w2b):
    grid_spec = pltpu.PrefetchScalarGridSpec(
        num_scalar_prefetch=1,
        grid=(NUM_TILES,),
        in_specs=[
            pl.BlockSpec((TILE, C), lambda i, te: (i, 0)),
            pl.BlockSpec((1, FFN, C), lambda i, te: (te[i], 0, 0)),
            pl.BlockSpec((1, C, FFN), lambda i, te: (te[i], 0, 0)),
        ],
        out_specs=pl.BlockSpec((TILE, C), lambda i, te: (i, 0)),
    )
    return pl.pallas_call(
        _mlp_body,
        grid_spec=grid_spec,
        out_shape=jax.ShapeDtypeStruct((PADDED, C), jnp.float32),
    )(meta, xs, w1b, w2b)


def _shared_body(x_ref, sg_ref, kpw_ref, kpb_ref, vpw_ref, vpb_ref, o_ref):
    x = x_ref[...]
    xn = x * jax.lax.rsqrt(jnp.mean(x * x, axis=1, keepdims=True) + EPS)
    g = jax.nn.sigmoid(jnp.sum(xn * sg_ref[...], axis=1, keepdims=True))
    x2 = xn * jax.lax.rsqrt(jnp.mean(xn * xn, axis=1, keepdims=True) + EPS)
    kk = jax.lax.dot_general(
        x2.astype(jnp.bfloat16), kpw_ref[...], (((1,), (1,)), ((), ())),
        preferred_element_type=jnp.float32) + kpb_ref[...]
    kk = jnp.square(jnp.maximum(kk, 0.0))
    sh = jax.lax.dot_general(
        kk.astype(jnp.bfloat16), vpw_ref[...], (((1,), (1,)), ((), ())),
        preferred_element_type=jnp.float32) + vpb_ref[...]
    o_ref[...] = g * sh


def _shared(xf, sg_w, kpw, kpb, vpw, vpb):
    rows = 256
    return pl.pallas_call(
        _shared_body,
        grid=(N // rows,),
        in_specs=[
            pl.BlockSpec((rows, C), lambda i: (i, 0)),
            pl.BlockSpec((1, C), lambda i: (0, 0)),
            pl.BlockSpec((FFN, C), lambda i: (0, 0)),
            pl.BlockSpec((1, FFN), lambda i: (0, 0)),
            pl.BlockSpec((C, FFN), lambda i: (0, 0)),
            pl.BlockSpec((1, C), lambda i: (0, 0)),
        ],
        out_specs=pl.BlockSpec((rows, C), lambda i: (i, 0)),
        out_shape=jax.ShapeDtypeStruct((N, C), jnp.float32),
    )(xf, sg_w, kpw, kpb, vpw, vpb)


def _combine_body(y0_ref, y1_ref, w_ref, shg_ref, o_ref):
    w = w_ref[...]
    o_ref[...] = (w[:, 0:1] * y0_ref[...] + w[:, 1:2] * y1_ref[...]
                  + shg_ref[...])


def _combine(y01, w, shg):
    rows = 256
    nblk = N // rows
    return pl.pallas_call(
        _combine_body,
        grid=(nblk,),
        in_specs=[
            pl.BlockSpec((rows, C), lambda i: (i, 0)),
            pl.BlockSpec((rows, C), lambda i: (i + nblk, 0)),
            pl.BlockSpec((rows, 2), lambda i: (i, 0)),
            pl.BlockSpec((rows, C), lambda i: (i, 0)),
        ],
        out_specs=pl.BlockSpec((rows, C), lambda i: (i, 0)),
        out_shape=jax.ShapeDtypeStruct((N, C), jnp.float32),
    )(y01, y01, w, shg)


def kernel(x, gate_w, W1, W2, sg_w, kp_w, kp_b, vp_w, vp_b):
    Bs, Ts, Cs = x.shape
    xf = x.reshape(Ts, Cs)
    xnb, dest, w, meta = _router(xf, gate_w)
    dest_cat = dest.T.reshape(2 * N)
    shg = _shared(xf, sg_w,
                  kp_w.astype(jnp.bfloat16), kp_b.reshape(1, FFN),
                  vp_w.astype(jnp.bfloat16), vp_b.reshape(1, C))
    xs = _sc_scatter(xnb, dest_cat)
    ys = _mlp(meta.reshape(MROWS), xs,
              W1.astype(jnp.bfloat16), W2.astype(jnp.bfloat16))
    y01 = _sc_gather(ys, dest_cat)
    out = _combine(y01, w, shg)
    return out.reshape(Bs, Ts, Cs)


# speedup vs baseline: 1.7868x; 1.2419x over previous
import functools

import jax
import jax.numpy as jnp
import numpy as np
from jax.experimental import pallas as pl
from jax.experimental.pallas import tpu as pltpu
from jax.experimental.pallas import tpu_sc as plsc

EPS = float(np.finfo(np.float32).eps)
N = 2048
C = 768
E = 8
FFN = 2048
TILE = 256
NUM_TILES = 24
MROWS = NUM_TILES + 8
PADDED = NUM_TILES * TILE
NSUB = 32
RPW = 2 * N // NSUB


def _router_body(x_ref, gate_ref, xnb_ref, dest_ref, w_ref, meta_ref):
    x = x_ref[...]
    xn = x * jax.lax.rsqrt(jnp.mean(x * x, axis=1, keepdims=True) + EPS)
    xnb_ref[...] = xn
    logits = jax.lax.dot_general(
        xn, gate_ref[...], (((1,), (1,)), ((), ())),
        preferred_element_type=jnp.float32)
    m1 = jnp.max(logits, axis=1, keepdims=True)
    ex = jnp.exp(logits - m1)
    scores = ex / jnp.sum(ex, axis=1, keepdims=True)
    iota_e = jax.lax.broadcasted_iota(jnp.int32, (N, E), 1)
    tri_e = (jax.lax.broadcasted_iota(jnp.int32, (E, E), 0)
             <= jax.lax.broadcasted_iota(jnp.int32, (E, E), 1)).astype(jnp.float32)
    eq1 = (logits == m1).astype(jnp.float32)
    cs1 = jax.lax.dot_general(eq1, tri_e, (((1,), (0,)), ((), ())),
                              preferred_element_type=jnp.float32)
    oh1 = eq1 * (cs1 == 1.0).astype(jnp.float32)
    masked = jnp.where(oh1 > 0.0, -jnp.inf, logits)
    m2 = jnp.max(masked, axis=1, keepdims=True)
    eq2 = (masked == m2).astype(jnp.float32)
    cs2 = jax.lax.dot_general(eq2, tri_e, (((1,), (0,)), ((), ())),
                              preferred_element_type=jnp.float32)
    oh2 = eq2 * (cs2 == 1.0).astype(jnp.float32)
    w1 = jnp.sum(scores * oh1, axis=1, keepdims=True)
    w2 = jnp.sum(scores * oh2, axis=1, keepdims=True)
    s = w1 + w2 + 1e-6
    iota2 = jax.lax.broadcasted_iota(jnp.int32, (N, 2), 1)
    w_ref[...] = jnp.where(iota2 == 0, w1 / s, w2 / s)
    e1 = jnp.sum(jnp.where(oh1 > 0.0, iota_e, 0), axis=1, keepdims=True)
    e2 = jnp.sum(jnp.where(oh2 > 0.0, iota_e, 0), axis=1, keepdims=True)
    iota16 = jax.lax.broadcasted_iota(jnp.int32, (N, 2 * E), 1)
    oh16 = (jnp.equal(iota16, e1) | jnp.equal(iota16, e2 + E)).astype(jnp.bfloat16)
    ltri = (jax.lax.broadcasted_iota(jnp.int32, (N, N), 1)
            < jax.lax.broadcasted_iota(jnp.int32, (N, N), 0)).astype(jnp.bfloat16)
    run = jax.lax.dot_general(ltri, oh16, (((1,), (0,)), ((), ())),
                              preferred_element_type=jnp.float32)
    cnt1 = jnp.sum(oh1, axis=0, keepdims=True)
    cnt2 = jnp.sum(oh2, axis=0, keepdims=True)
    cnt = cnt1 + cnt2
    ptiles = jnp.ceil(cnt * (1.0 / TILE))
    stri_e = (jax.lax.broadcasted_iota(jnp.int32, (E, E), 0)
              < jax.lax.broadcasted_iota(jnp.int32, (E, E), 1)).astype(jnp.float32)
    base_tile = jax.lax.dot_general(ptiles, stri_e, (((1,), (0,)), ((), ())),
                                    preferred_element_type=jnp.float32)
    base_rows = base_tile * float(TILE)
    rank1 = jnp.sum(run[:, :E] * oh1, axis=1, keepdims=True)
    rank2 = (jnp.sum(run[:, E:] * oh2, axis=1, keepdims=True)
             + jnp.sum(cnt1 * oh2, axis=1, keepdims=True))
    dest1 = jnp.sum(base_rows * oh1, axis=1, keepdims=True) + rank1
    dest2 = jnp.sum(base_rows * oh2, axis=1, keepdims=True) + rank2
    dest_ref[...] = jnp.where(iota2 == 0, dest1, dest2).astype(jnp.int32)
    trow = jax.lax.broadcasted_iota(jnp.int32, (MROWS, E), 0).astype(jnp.float32)
    te = jnp.sum((base_tile <= trow).astype(jnp.float32), axis=1, keepdims=True) - 1.0
    n_active = jnp.sum(ptiles)
    mrow = jax.lax.broadcasted_iota(jnp.int32, (MROWS, 1), 0)
    meta_ref[...] = jnp.where(mrow == NUM_TILES, n_active, te).astype(jnp.int32)


def _router(xf, gate_w):
    return pl.pallas_call(
        _router_body,
        out_shape=(
            jax.ShapeDtypeStruct((N, C), jnp.float32),
            jax.ShapeDtypeStruct((N, 2), jnp.int32),
            jax.ShapeDtypeStruct((N, 2), jnp.float32),
            jax.ShapeDtypeStruct((MROWS, 1), jnp.int32),
        ),
    )(xf, gate_w)


def _sc_scatter(xnb, dest_cat):
    mesh = plsc.VectorSubcoreMesh(core_axis_name="c", subcore_axis_name="s")

    @functools.partial(
        pl.kernel,
        out_type=jax.ShapeDtypeStruct((PADDED, C), jnp.float32),
        mesh=mesh,
        scratch_types=[
            pltpu.VMEM((RPW,), jnp.int32),
            pltpu.VMEM((RPW, C), jnp.float32),
            pltpu.SemaphoreType.DMA,
        ])
    def k(x_hbm, i_hbm, o_hbm, idx_v, rows_v, sem):
        wid = jax.lax.axis_index("s") * 2 + jax.lax.axis_index("c")
        base = wid * RPW
        pltpu.sync_copy(i_hbm.at[pl.ds(base, RPW)], idx_v)
        pltpu.sync_copy(x_hbm.at[pl.ds(base % N, RPW)], rows_v)
        pltpu.async_copy(rows_v, o_hbm.at[idx_v], sem).wait()

    return k(xnb, dest_cat)


def _sc_gather(ys, dest_cat):
    mesh = plsc.VectorSubcoreMesh(core_axis_name="c", subcore_axis_name="s")

    @functools.partial(
        pl.kernel,
        out_type=jax.ShapeDtypeStruct((2 * N, C), jnp.float32),
        mesh=mesh,
        scratch_types=[
            pltpu.VMEM((RPW,), jnp.int32),
            pltpu.VMEM((RPW, C), jnp.float32),
            pltpu.SemaphoreType.DMA,
        ])
    def k(y_hbm, i_hbm, o_hbm, idx_v, rows_v, sem):
        wid = jax.lax.axis_index("s") * 2 + jax.lax.axis_index("c")
        base = wid * RPW
        pltpu.sync_copy(i_hbm.at[pl.ds(base, RPW)], idx_v)
        pltpu.async_copy(y_hbm.at[idx_v], rows_v, sem).wait()
        pltpu.sync_copy(rows_v, o_hbm.at[pl.ds(base, RPW)])

    return k(ys, dest_cat)


def _mlp_body(te_ref, xs_ref, w1_ref, w2_ref, ys_ref):
    i = pl.program_id(0)

    @pl.when(i < te_ref[NUM_TILES])
    def _():
        h = jax.lax.dot_general(
            xs_ref[...].astype(jnp.bfloat16),
            w1_ref[0].astype(jnp.bfloat16), (((1,), (1,)), ((), ())),
            preferred_element_type=jnp.float32)
        h = jnp.square(jnp.maximum(h, 0.0)).astype(jnp.bfloat16)
        ys_ref[...] = jax.lax.dot_general(
            h, w2_ref[0].astype(jnp.bfloat16), (((1,), (1,)), ((), ())),
            preferred_element_type=jnp.float32)


def _mlp(meta, xs, w1b, w2b):
    grid_spec = pltpu.PrefetchScalarGridSpec(
        num_scalar_prefetch=1,
        grid=(NUM_TILES,),
        in_specs=[
            pl.BlockSpec((TILE, C), lambda i, te: (i, 0)),
            pl.BlockSpec((1, FFN, C), lambda i, te: (te[i], 0, 0)),
            pl.BlockSpec((1, C, FFN), lambda i, te: (te[i], 0, 0)),
        ],
        out_specs=pl.BlockSpec((TILE, C), lambda i, te: (i, 0)),
    )
    return pl.pallas_call(
        _mlp_body,
        grid_spec=grid_spec,
        out_shape=jax.ShapeDtypeStruct((PADDED, C), jnp.float32),
    )(meta, xs, w1b, w2b)


def _shared_body(x_ref, sg_ref, kpw_ref, kpb_ref, vpw_ref, vpb_ref, o_ref):
    x = x_ref[...]
    xn = x * jax.lax.rsqrt(jnp.mean(x * x, axis=1, keepdims=True) + EPS)
    g = jax.nn.sigmoid(jnp.sum(xn * sg_ref[...], axis=1, keepdims=True))
    x2 = xn * jax.lax.rsqrt(jnp.mean(xn * xn, axis=1, keepdims=True) + EPS)
    kk = jax.lax.dot_general(
        x2.astype(jnp.bfloat16), kpw_ref[...].astype(jnp.bfloat16),
        (((1,), (1,)), ((), ())),
        preferred_element_type=jnp.float32) + kpb_ref[...]
    kk = jnp.square(jnp.maximum(kk, 0.0))
    sh = jax.lax.dot_general(
        kk.astype(jnp.bfloat16), vpw_ref[...].astype(jnp.bfloat16),
        (((1,), (1,)), ((), ())),
        preferred_element_type=jnp.float32) + vpb_ref[...]
    o_ref[...] = g * sh


def _shared(xf, sg_w, kpw, kpb, vpw, vpb):
    rows = 256
    return pl.pallas_call(
        _shared_body,
        grid=(N // rows,),
        in_specs=[
            pl.BlockSpec((rows, C), lambda i: (i, 0)),
            pl.BlockSpec((1, C), lambda i: (0, 0)),
            pl.BlockSpec((FFN, C), lambda i: (0, 0)),
            pl.BlockSpec((1, FFN), lambda i: (0, 0)),
            pl.BlockSpec((C, FFN), lambda i: (0, 0)),
            pl.BlockSpec((1, C), lambda i: (0, 0)),
        ],
        out_specs=pl.BlockSpec((rows, C), lambda i: (i, 0)),
        out_shape=jax.ShapeDtypeStruct((N, C), jnp.float32),
    )(xf, sg_w, kpw, kpb, vpw, vpb)


def _combine_body(y0_ref, y1_ref, w_ref, shg_ref, o_ref):
    w = w_ref[...]
    o_ref[...] = (w[:, 0:1] * y0_ref[...] + w[:, 1:2] * y1_ref[...]
                  + shg_ref[...])


def _combine(y01, w, shg):
    rows = 256
    nblk = N // rows
    return pl.pallas_call(
        _combine_body,
        grid=(nblk,),
        in_specs=[
            pl.BlockSpec((rows, C), lambda i: (i, 0)),
            pl.BlockSpec((rows, C), lambda i: (i + nblk, 0)),
            pl.BlockSpec((rows, 2), lambda i: (i, 0)),
            pl.BlockSpec((rows, C), lambda i: (i, 0)),
        ],
        out_specs=pl.BlockSpec((rows, C), lambda i: (i, 0)),
        out_shape=jax.ShapeDtypeStruct((N, C), jnp.float32),
    )(y01, y01, w, shg)


def kernel(x, gate_w, W1, W2, sg_w, kp_w, kp_b, vp_w, vp_b):
    Bs, Ts, Cs = x.shape
    xf = x.reshape(Ts, Cs)
    xnb, dest, w, meta = _router(xf, gate_w)
    dest_cat = dest.T.reshape(2 * N)
    shg = _shared(xf, sg_w, kp_w, kp_b.reshape(1, FFN),
                  vp_w, vp_b.reshape(1, C))
    xs = _sc_scatter(xnb, dest_cat)
    ys = _mlp(meta.reshape(MROWS), xs, W1, W2)
    y01 = _sc_gather(ys, dest_cat)
    out = _combine(y01, w, shg)
    return out.reshape(Bs, Ts, Cs)
